# baseline (speedup 1.0000x reference)
import jax
import jax.numpy as jnp
from jax import lax
from jax.experimental import pallas as pl
from jax.experimental.pallas import tpu as pltpu
from jax.experimental.pallas import tpu_sc as plsc


def _sc_body(p_hbm, wt_hbm, out_hbm, p_v, wt_v, out_v, sem):
    sid = lax.axis_index("s")

    @pl.when(sid == 0)
    def _():
        c1 = pltpu.async_copy(p_hbm, p_v, sem)
        c2 = pltpu.async_copy(wt_hbm, wt_v, sem)
        c1.wait()
        c2.wait()
        xvec = p_v[pl.ds(0, 16)]
        bvec = p_v[pl.ds(16, 16)]
        tokvec = p_v[pl.ds(32, 16)]
        acc0 = bvec
        acc1 = bvec
        for k in range(8):
            col = wt_v[k, :]
            acc0 = acc0 + tokvec[k] * col
            acc1 = acc1 + xvec[8 + k] * col
        out_v[0, :] = acc0
        out_v[1, :] = acc1
        pltpu.sync_copy(out_v, out_hbm)


def kernel(x, W, b, token):
    mesh = plsc.VectorSubcoreMesh(
        core_axis_name="c", subcore_axis_name="s", num_cores=1
    )
    packed = jnp.concatenate(
        [x.reshape(-1), b, token, jnp.zeros((8,), jnp.float32)]
    )
    f = pl.kernel(
        _sc_body,
        out_type=jax.ShapeDtypeStruct((2, 16), jnp.float32),
        mesh=mesh,
        scratch_types=[
            pltpu.VMEM((48,), jnp.float32),
            pltpu.VMEM((8, 16), jnp.float32),
            pltpu.VMEM((2, 16), jnp.float32),
            pltpu.SemaphoreType.DMA,
        ],
    )
    return f(packed, W.T)
